# Initial kernel scaffold; baseline (speedup 1.0000x reference)
import functools

import jax
import jax.numpy as jnp
from jax import lax
from jax.experimental import pallas as pl
from jax.experimental.pallas import tpu as pltpu
from jax.experimental.pallas import tpu_sc as plsc

N = 10000
E = 320000
F = 128

NC = 2
NS = 16
NW = NC * NS
C = 100
K = E // (NW * C)
RPT = N // NS
ZR = 125

_f32 = jnp.float32
_mesh = plsc.VectorSubcoreMesh(core_axis_name="c", subcore_axis_name="s")


def _wid(c, s):
    return c * NS + s


def _deg_body(edge_hbm, ones_hbm, z_hbm, out_hbm,
              src_v, dst_v, ones_v, zst, degs_sh, degd_sh):
    c = lax.axis_index("c")
    s = lax.axis_index("s")
    w = _wid(c, s)
    base = s * RPT

    pltpu.sync_copy(edge_hbm.at[0, w], src_v)
    pltpu.sync_copy(edge_hbm.at[1, w], dst_v)
    pltpu.sync_copy(ones_hbm, ones_v)
    pltpu.sync_copy(z_hbm, zst)
    pltpu.sync_copy(zst, degs_sh.at[pl.ds(base, RPT)])
    pltpu.sync_copy(zst, degd_sh.at[pl.ds(base, RPT)])
    plsc.subcore_barrier()

    def body(j, carry):
        pltpu.sync_copy(ones_v, degs_sh.at[src_v.at[j]], add=True)
        pltpu.sync_copy(ones_v, degd_sh.at[dst_v.at[j]], add=True)
        return carry

    lax.fori_loop(0, K, body, 0)
    plsc.subcore_barrier()

    pltpu.sync_copy(degs_sh.at[pl.ds(base, RPT)], zst)
    pltpu.sync_copy(zst, out_hbm.at[c, 0, pl.ds(base, RPT)])
    pltpu.sync_copy(degd_sh.at[pl.ds(base, RPT)], zst)
    pltpu.sync_copy(zst, out_hbm.at[c, 1, pl.ds(base, RPT)])


_deg_call = pl.kernel(
    _deg_body,
    out_type=jax.ShapeDtypeStruct((2, 2, N, 16), _f32),
    mesh=_mesh,
    scratch_types=[
        pltpu.VMEM((K, C), jnp.int32),
        pltpu.VMEM((K, C), jnp.int32),
        pltpu.VMEM((C, 16), _f32),
        pltpu.VMEM((RPT, 16), _f32),
        pltpu.VMEM_SHARED((N, 16), _f32),
        pltpu.VMEM_SHARED((N, 16), _f32),
    ],
)


def _agg_body(hs_hbm, edge_hbm, z_hbm, out_hbm,
              src_v, dst_v, rows0, rows1, zst, acc_sh, sem0, sem1):
    c = lax.axis_index("c")
    s = lax.axis_index("s")
    w = _wid(c, s)
    base = s * RPT

    pltpu.sync_copy(edge_hbm.at[0, w], src_v)
    pltpu.sync_copy(edge_hbm.at[1, w], dst_v)
    pltpu.sync_copy(z_hbm, zst)
    for i in range(RPT // ZR):
        pltpu.sync_copy(zst, acc_sh.at[pl.ds(base + i * ZR, ZR)])
    pltpu.async_copy(hs_hbm.at[src_v.at[0]], rows0, sem0)
    pltpu.async_copy(hs_hbm.at[src_v.at[1]], rows1, sem1)
    plsc.subcore_barrier()

    def chunk(j, buf, sem):
        pltpu.make_async_copy(hs_hbm.at[src_v.at[j]], buf, sem).wait()
        pltpu.sync_copy(buf, acc_sh.at[dst_v.at[j]], add=True)

        @pl.when(j + 2 < K)
        def _():
            pltpu.async_copy(hs_hbm.at[src_v.at[j + 2]], buf, sem)

    def body(t, carry):
        chunk(2 * t, rows0, sem0)
        chunk(2 * t + 1, rows1, sem1)
        return carry

    lax.fori_loop(0, K // 2, body, 0)
    plsc.subcore_barrier()

    for i in range(RPT // ZR):
        pltpu.sync_copy(acc_sh.at[pl.ds(base + i * ZR, ZR)], zst)
        pltpu.sync_copy(zst, out_hbm.at[c, pl.ds(base + i * ZR, ZR)])


_agg_call = pl.kernel(
    _agg_body,
    out_type=jax.ShapeDtypeStruct((2, N, F), _f32),
    mesh=_mesh,
    scratch_types=[
        pltpu.VMEM((K, C), jnp.int32),
        pltpu.VMEM((K, C), jnp.int32),
        pltpu.VMEM((C, F), _f32),
        pltpu.VMEM((C, F), _f32),
        pltpu.VMEM((ZR, F), _f32),
        pltpu.VMEM_SHARED((N, F), _f32),
        pltpu.SemaphoreType.DMA,
        pltpu.SemaphoreType.DMA,
    ],
)


BN = 1000
GRID = N // BN


def _embed_body(x_ref, w_ref, b_ref, o_ref):
    o_ref[...] = jnp.dot(x_ref[...], w_ref[...],
                         preferred_element_type=_f32) + b_ref[...]


def _embed(x, W, b):
    return pl.pallas_call(
        _embed_body,
        grid=(GRID,),
        in_specs=[
            pl.BlockSpec((BN, F), lambda i: (i, 0)),
            pl.BlockSpec((F, F), lambda i: (0, 0)),
            pl.BlockSpec((1, F), lambda i: (0, 0)),
        ],
        out_specs=pl.BlockSpec((BN, F), lambda i: (i, 0)),
        out_shape=jax.ShapeDtypeStruct((N, F), _f32),
    )(x, W, b.reshape(1, F))


def _scale_body(h_ref, d_ref, hs_ref, io_ref, ii_ref):
    d = d_ref[...]
    do = d[0, :, 0:1] + d[2, :, 0:1]
    di = d[1, :, 0:1] + d[3, :, 0:1]
    io = jnp.where(do > 0, lax.rsqrt(do), 0.0)
    ii = jnp.where(di > 0, lax.rsqrt(di), 0.0)
    hs_ref[...] = h_ref[...] * io
    io_ref[...] = io
    ii_ref[...] = ii


def _scale(h, deg):
    return pl.pallas_call(
        _scale_body,
        grid=(GRID,),
        in_specs=[
            pl.BlockSpec((BN, F), lambda i: (i, 0)),
            pl.BlockSpec((4, BN, 16), lambda i: (0, i, 0)),
        ],
        out_specs=[
            pl.BlockSpec((BN, F), lambda i: (i, 0)),
            pl.BlockSpec((BN, 1), lambda i: (i, 0)),
            pl.BlockSpec((BN, 1), lambda i: (i, 0)),
        ],
        out_shape=[
            jax.ShapeDtypeStruct((N, F), _f32),
            jax.ShapeDtypeStruct((N, 1), _f32),
            jax.ShapeDtypeStruct((N, 1), _f32),
        ],
    )(h, deg)


def _layer_body(p_ref, ii_ref, sc_ref, w_ref, b_ref, o_ref):
    p = p_ref[...]
    m = (p[0] + p[1]) * ii_ref[...]
    h = jnp.maximum(jnp.dot(m, w_ref[...], preferred_element_type=_f32)
                    + b_ref[...], 0.0)
    o_ref[...] = h * sc_ref[...]


def _layer(p, ii, scale, W, b):
    return pl.pallas_call(
        _layer_body,
        grid=(GRID,),
        in_specs=[
            pl.BlockSpec((2, BN, F), lambda i: (0, i, 0)),
            pl.BlockSpec((BN, 1), lambda i: (i, 0)),
            pl.BlockSpec((BN, 1), lambda i: (i, 0)),
            pl.BlockSpec((F, F), lambda i: (0, 0)),
            pl.BlockSpec((1, F), lambda i: (0, 0)),
        ],
        out_specs=pl.BlockSpec((BN, F), lambda i: (i, 0)),
        out_shape=jax.ShapeDtypeStruct((N, F), _f32),
    )(p, ii, scale, W, b.reshape(1, F))


def _head_body(h_ref, w1_ref, b1_ref, w2_ref, b2_ref, w3_ref, b3_ref,
               o_ref, acc_ref):
    i = pl.program_id(0)
    s = jnp.sum(h_ref[...], axis=0, keepdims=True)

    @pl.when(i == 0)
    def _():
        acc_ref[...] = s

    @pl.when(i > 0)
    def _():
        acc_ref[...] = acc_ref[...] + s

    @pl.when(i == GRID - 1)
    def _():
        g = acc_ref[...] * (1.0 / N)
        o = jnp.maximum(jnp.dot(g, w1_ref[...],
                                preferred_element_type=_f32) + b1_ref[...], 0.0)
        o = jnp.maximum(jnp.dot(o, w2_ref[...],
                                preferred_element_type=_f32) + b2_ref[...], 0.0)
        o_ref[...] = jnp.dot(o, w3_ref[...],
                             preferred_element_type=_f32) + b3_ref[...]


def _head(h, Wo1, bo1, Wo2, bo2, Wo3, bo3):
    H2, H4 = F // 2, F // 4
    return pl.pallas_call(
        _head_body,
        grid=(GRID,),
        in_specs=[
            pl.BlockSpec((BN, F), lambda i: (i, 0)),
            pl.BlockSpec((F, H2), lambda i: (0, 0)),
            pl.BlockSpec((1, H2), lambda i: (0, 0)),
            pl.BlockSpec((H2, H4), lambda i: (0, 0)),
            pl.BlockSpec((1, H4), lambda i: (0, 0)),
            pl.BlockSpec((H4, 1), lambda i: (0, 0)),
            pl.BlockSpec((1, 1), lambda i: (0, 0)),
        ],
        out_specs=pl.BlockSpec((1, 1), lambda i: (0, 0)),
        out_shape=jax.ShapeDtypeStruct((1, 1), _f32),
        scratch_shapes=[pltpu.VMEM((1, F), _f32)],
    )(h, Wo1, bo1.reshape(1, H2), Wo2, bo2.reshape(1, H4),
      Wo3, bo3.reshape(1, 1))


def kernel(x, edge_index, W_atom, b_atom, W1, b1, W2, b2, W3, b3,
           Wo1, bo1, Wo2, bo2, Wo3, bo3):
    e3 = edge_index.astype(jnp.int32).reshape(2, NW, K, C)
    ones16 = jnp.ones((C, 16), _f32)
    z16 = jnp.zeros((RPT, 16), _f32)
    zrow = jnp.zeros((ZR, F), _f32)

    deg = _deg_call(e3, ones16, z16).reshape(4, N, 16)
    h = _embed(x, W_atom, b_atom)
    hs, io, ii = _scale(h, deg)

    ones_n = jnp.ones((N, 1), _f32)
    for W, b, sc in ((W1, b1, io), (W2, b2, io), (W3, b3, ones_n)):
        p = _agg_call(hs, e3, zrow)
        hs = _layer(p, ii, sc, W, b)

    out = _head(hs, Wo1, bo1, Wo2, bo2, Wo3, bo3)
    return out.reshape(1)


# speedup vs baseline: 8.3734x; 8.3734x over previous
import functools

import jax
import jax.numpy as jnp
from jax import lax
from jax.experimental import pallas as pl
from jax.experimental.pallas import tpu as pltpu
from jax.experimental.pallas import tpu_sc as plsc

N = 10000
E = 320000
F = 128

NC = 2
NS = 16
NW = NC * NS
C = 100
K = E // (NW * C)
RPT = N // NS
ZR = 125

_f32 = jnp.float32
_mesh = plsc.VectorSubcoreMesh(core_axis_name="c", subcore_axis_name="s")
_sc_params = pltpu.CompilerParams(use_tc_tiling_on_sc=False)


def _wid(c, s):
    return c * NS + s


def _deg_body(edge_hbm, ones_hbm, z_hbm, out_hbm,
              src_v, dst_v, ones_v, degs_sh, degd_sh):
    c = lax.axis_index("c")
    s = lax.axis_index("s")
    w = _wid(c, s)

    pltpu.sync_copy(edge_hbm.at[0, w], src_v)
    pltpu.sync_copy(edge_hbm.at[1, w], dst_v)
    pltpu.sync_copy(ones_hbm, ones_v)

    @pl.when(s == 0)
    def _():
        pltpu.sync_copy(z_hbm, degs_sh)
        pltpu.sync_copy(z_hbm, degd_sh)

    plsc.subcore_barrier()

    def body(j, carry):
        pltpu.sync_copy(ones_v, degs_sh.at[src_v.at[j]], add=True)
        pltpu.sync_copy(ones_v, degd_sh.at[dst_v.at[j]], add=True)
        return carry

    lax.fori_loop(0, K, body, 0)
    plsc.subcore_barrier()

    @pl.when(s == 0)
    def _():
        pltpu.sync_copy(degs_sh, out_hbm.at[c, 0])
        pltpu.sync_copy(degd_sh, out_hbm.at[c, 1])


_deg_call = pl.kernel(
    _deg_body,
    out_type=jax.ShapeDtypeStruct((2, 2, N, 16), _f32),
    mesh=_mesh,
    scratch_types=[
        pltpu.VMEM((K, C), jnp.int32),
        pltpu.VMEM((K, C), jnp.int32),
        pltpu.VMEM((C, 16), _f32),
        pltpu.VMEM_SHARED((N, 16), _f32),
        pltpu.VMEM_SHARED((N, 16), _f32),
    ],
    compiler_params=_sc_params,
)


HF = F // 2
K2 = E // (NS * C)


def _agg_body(hs_hbm, edge_hbm, z_hbm, out_hbm,
              src_v, dst_v, rows0, rows1, acc_sh, sem0, sem1):
    c = lax.axis_index("c")
    s = lax.axis_index("s")
    hsc = hs_hbm.at[c]

    pltpu.sync_copy(edge_hbm.at[0, s], src_v)
    pltpu.sync_copy(edge_hbm.at[1, s], dst_v)

    @pl.when(s == 0)
    def _():
        pltpu.sync_copy(z_hbm, acc_sh)

    pltpu.async_copy(hsc.at[src_v.at[0]], rows0, sem0)
    pltpu.async_copy(hsc.at[src_v.at[1]], rows1, sem1)
    plsc.subcore_barrier()

    def chunk(j, buf, sem):
        pltpu.make_async_copy(hsc.at[src_v.at[j]], buf, sem).wait()
        pltpu.sync_copy(buf, acc_sh.at[dst_v.at[j]], add=True)

        @pl.when(j + 2 < K2)
        def _():
            pltpu.async_copy(hsc.at[src_v.at[j + 2]], buf, sem)

    def body(t, carry):
        chunk(2 * t, rows0, sem0)
        chunk(2 * t + 1, rows1, sem1)
        return carry

    lax.fori_loop(0, K2 // 2, body, 0)
    plsc.subcore_barrier()

    @pl.when(s == 0)
    def _():
        pltpu.sync_copy(acc_sh, out_hbm.at[c])


_agg_call = pl.kernel(
    _agg_body,
    out_type=jax.ShapeDtypeStruct((2, N, HF), _f32),
    mesh=_mesh,
    scratch_types=[
        pltpu.VMEM((K2, C), jnp.int32),
        pltpu.VMEM((K2, C), jnp.int32),
        pltpu.VMEM((C, HF), _f32),
        pltpu.VMEM((C, HF), _f32),
        pltpu.VMEM_SHARED((N, HF), _f32),
        pltpu.SemaphoreType.DMA,
        pltpu.SemaphoreType.DMA,
    ],
    compiler_params=_sc_params,
)


BN = 1000
GRID = N // BN


def _embed_body(x_ref, w_ref, b_ref, o_ref):
    o_ref[...] = jnp.dot(x_ref[...], w_ref[...],
                         preferred_element_type=_f32) + b_ref[...]


def _embed(x, W, b):
    return pl.pallas_call(
        _embed_body,
        grid=(GRID,),
        in_specs=[
            pl.BlockSpec((BN, F), lambda i: (i, 0)),
            pl.BlockSpec((F, F), lambda i: (0, 0)),
            pl.BlockSpec((1, F), lambda i: (0, 0)),
        ],
        out_specs=pl.BlockSpec((BN, F), lambda i: (i, 0)),
        out_shape=jax.ShapeDtypeStruct((N, F), _f32),
    )(x, W, b.reshape(1, F))


def _split_store(hs_ref, v):
    hs_ref[0, :, :] = v[:, :HF]
    hs_ref[1, :, :] = v[:, HF:]


def _scale_body(h_ref, d_ref, hs_ref, io_ref, ii_ref):
    d = d_ref[...]
    do = d[0, :, 0:1] + d[2, :, 0:1]
    di = d[1, :, 0:1] + d[3, :, 0:1]
    io = jnp.where(do > 0, lax.rsqrt(do), 0.0)
    ii = jnp.where(di > 0, lax.rsqrt(di), 0.0)
    _split_store(hs_ref, h_ref[...] * io)
    io_ref[...] = io
    ii_ref[...] = ii


def _scale(h, deg):
    return pl.pallas_call(
        _scale_body,
        grid=(GRID,),
        in_specs=[
            pl.BlockSpec((BN, F), lambda i: (i, 0)),
            pl.BlockSpec((4, BN, 16), lambda i: (0, i, 0)),
        ],
        out_specs=[
            pl.BlockSpec((2, BN, HF), lambda i: (0, i, 0)),
            pl.BlockSpec((BN, 1), lambda i: (i, 0)),
            pl.BlockSpec((BN, 1), lambda i: (i, 0)),
        ],
        out_shape=[
            jax.ShapeDtypeStruct((2, N, HF), _f32),
            jax.ShapeDtypeStruct((N, 1), _f32),
            jax.ShapeDtypeStruct((N, 1), _f32),
        ],
    )(h, deg)


def _layer_body(p_ref, ii_ref, sc_ref, w_ref, b_ref, o_ref):
    p = p_ref[...]
    m = jnp.concatenate([p[0], p[1]], axis=1) * ii_ref[...]
    h = jnp.maximum(jnp.dot(m, w_ref[...], preferred_element_type=_f32)
                    + b_ref[...], 0.0)
    _split_store(o_ref, h * sc_ref[...])


def _layer(p, ii, scale, W, b):
    return pl.pallas_call(
        _layer_body,
        grid=(GRID,),
        in_specs=[
            pl.BlockSpec((2, BN, HF), lambda i: (0, i, 0)),
            pl.BlockSpec((BN, 1), lambda i: (i, 0)),
            pl.BlockSpec((BN, 1), lambda i: (i, 0)),
            pl.BlockSpec((F, F), lambda i: (0, 0)),
            pl.BlockSpec((1, F), lambda i: (0, 0)),
        ],
        out_specs=pl.BlockSpec((2, BN, HF), lambda i: (0, i, 0)),
        out_shape=jax.ShapeDtypeStruct((2, N, HF), _f32),
    )(p, ii, scale, W, b.reshape(1, F))


def _head_body(h_ref, w1_ref, b1_ref, w2_ref, b2_ref, w3_ref, b3_ref,
               o_ref, acc_ref):
    i = pl.program_id(0)
    hb = h_ref[...]
    s = jnp.concatenate(
        [jnp.sum(hb[0], axis=0, keepdims=True),
         jnp.sum(hb[1], axis=0, keepdims=True)], axis=1)

    @pl.when(i == 0)
    def _():
        acc_ref[...] = s

    @pl.when(i > 0)
    def _():
        acc_ref[...] = acc_ref[...] + s

    @pl.when(i == GRID - 1)
    def _():
        g = acc_ref[...] * (1.0 / N)
        o = jnp.maximum(jnp.dot(g, w1_ref[...],
                                preferred_element_type=_f32) + b1_ref[...], 0.0)
        o = jnp.maximum(jnp.dot(o, w2_ref[...],
                                preferred_element_type=_f32) + b2_ref[...], 0.0)
        o_ref[...] = jnp.dot(o, w3_ref[...],
                             preferred_element_type=_f32) + b3_ref[...]


def _head(h, Wo1, bo1, Wo2, bo2, Wo3, bo3):
    H2, H4 = F // 2, F // 4
    return pl.pallas_call(
        _head_body,
        grid=(GRID,),
        in_specs=[
            pl.BlockSpec((2, BN, HF), lambda i: (0, i, 0)),
            pl.BlockSpec((F, H2), lambda i: (0, 0)),
            pl.BlockSpec((1, H2), lambda i: (0, 0)),
            pl.BlockSpec((H2, H4), lambda i: (0, 0)),
            pl.BlockSpec((1, H4), lambda i: (0, 0)),
            pl.BlockSpec((H4, 1), lambda i: (0, 0)),
            pl.BlockSpec((1, 1), lambda i: (0, 0)),
        ],
        out_specs=pl.BlockSpec((1, 1), lambda i: (0, 0)),
        out_shape=jax.ShapeDtypeStruct((1, 1), _f32),
        scratch_shapes=[pltpu.VMEM((1, F), _f32)],
    )(h, Wo1, bo1.reshape(1, H2), Wo2, bo2.reshape(1, H4),
      Wo3, bo3.reshape(1, 1))


def kernel(x, edge_index, W_atom, b_atom, W1, b1, W2, b2, W3, b3,
           Wo1, bo1, Wo2, bo2, Wo3, bo3):
    ei = edge_index.astype(jnp.int32)
    e3d = ei.reshape(2, NW, K, C)
    e3a = ei.reshape(2, NS, K2, C)
    ones16 = jnp.ones((C, 16), _f32)
    z16 = jnp.zeros((N, 16), _f32)
    zrow = jnp.zeros((N, HF), _f32)

    deg = _deg_call(e3d, ones16, z16).reshape(4, N, 16)
    h = _embed(x, W_atom, b_atom)
    hs, io, ii = _scale(h, deg)

    ones_n = jnp.ones((N, 1), _f32)
    for W, b, sc in ((W1, b1, io), (W2, b2, io), (W3, b3, ones_n)):
        p = _agg_call(hs, e3a, zrow)
        hs = _layer(p, ii, sc, W, b)

    out = _head(hs, Wo1, bo1, Wo2, bo2, Wo3, bo3)
    return out.reshape(1)


# speedup vs baseline: 9.2523x; 1.1050x over previous
import functools

import jax
import jax.numpy as jnp
from jax import lax
from jax.experimental import pallas as pl
from jax.experimental.pallas import tpu as pltpu
from jax.experimental.pallas import tpu_sc as plsc

N = 10000
E = 320000
F = 128

NC = 2
NS = 16
NW = NC * NS
C = 100
K = E // (NW * C)
RPT = N // NS
ZR = 125

_f32 = jnp.float32
_mesh = plsc.VectorSubcoreMesh(core_axis_name="c", subcore_axis_name="s")
_sc_params = pltpu.CompilerParams(use_tc_tiling_on_sc=False)


def _wid(c, s):
    return c * NS + s


def _deg_body(edge_hbm, ones_hbm, z_hbm, out_hbm,
              src_v, dst_v, ones_v, degs_sh, degd_sh):
    c = lax.axis_index("c")
    s = lax.axis_index("s")
    w = _wid(c, s)

    pltpu.sync_copy(edge_hbm.at[0, w], src_v)
    pltpu.sync_copy(edge_hbm.at[1, w], dst_v)
    pltpu.sync_copy(ones_hbm, ones_v)

    @pl.when(s == 0)
    def _():
        pltpu.sync_copy(z_hbm, degs_sh)
        pltpu.sync_copy(z_hbm, degd_sh)

    plsc.subcore_barrier()

    def body(j, carry):
        pltpu.sync_copy(ones_v, degs_sh.at[src_v.at[j]], add=True)
        pltpu.sync_copy(ones_v, degd_sh.at[dst_v.at[j]], add=True)
        return carry

    lax.fori_loop(0, K, body, 0)
    plsc.subcore_barrier()

    @pl.when(s == 0)
    def _():
        pltpu.sync_copy(degs_sh, out_hbm.at[c, 0])
        pltpu.sync_copy(degd_sh, out_hbm.at[c, 1])


_deg_call = pl.kernel(
    _deg_body,
    out_type=jax.ShapeDtypeStruct((2, 2, N, 16), _f32),
    mesh=_mesh,
    scratch_types=[
        pltpu.VMEM((K, C), jnp.int32),
        pltpu.VMEM((K, C), jnp.int32),
        pltpu.VMEM((C, 16), _f32),
        pltpu.VMEM_SHARED((N, 16), _f32),
        pltpu.VMEM_SHARED((N, 16), _f32),
    ],
    compiler_params=_sc_params,
)


HF = F // 2
CA = 125
K2 = E // (NS * CA)
NBUF = 4
HALF = NBUF // 2


def _agg_body(hs_hbm, edge_hbm, z_hbm, out_hbm,
              src_v, dst_v, rows, gsem, ssem, acc_sh):
    c = lax.axis_index("c")
    s = lax.axis_index("s")
    hsc = hs_hbm.at[c]

    pltpu.sync_copy(edge_hbm.at[0, s], src_v)
    pltpu.sync_copy(edge_hbm.at[1, s], dst_v)

    @pl.when(s == 0)
    def _():
        pltpu.sync_copy(z_hbm, acc_sh)

    for b in range(NBUF):
        pltpu.async_copy(hsc.at[src_v.at[b]], rows.at[b], gsem[b])
    plsc.subcore_barrier()

    def chunk(j, b):
        pltpu.make_async_copy(hsc.at[src_v.at[j]], rows.at[b], gsem[b]).wait()
        pltpu.async_copy(rows.at[b], acc_sh.at[dst_v.at[j]], ssem[b],
                         add=True)
        bq = (b - HALF) % NBUF

        @pl.when(jnp.logical_and(j >= HALF, j - HALF + NBUF < K2))
        def _():
            pltpu.make_async_copy(rows.at[bq], acc_sh.at[dst_v.at[0]],
                                  ssem[bq]).wait()
            pltpu.async_copy(hsc.at[src_v.at[j - HALF + NBUF]], rows.at[bq],
                             gsem[bq])

    def body(t, carry):
        j = t * NBUF
        for b in range(NBUF):
            chunk(j + b, b)
        return carry

    lax.fori_loop(0, K2 // NBUF, body, 0)
    for b in range(NBUF):
        pltpu.make_async_copy(rows.at[b], acc_sh.at[dst_v.at[0]],
                              ssem[b]).wait()
    plsc.subcore_barrier()

    @pl.when(s == 0)
    def _():
        pltpu.sync_copy(acc_sh, out_hbm.at[c])


_agg_call = pl.kernel(
    _agg_body,
    out_type=jax.ShapeDtypeStruct((2, N, HF), _f32),
    mesh=_mesh,
    scratch_types=[
        pltpu.VMEM((K2, CA), jnp.int32),
        pltpu.VMEM((K2, CA), jnp.int32),
        pltpu.VMEM((NBUF, CA, HF), _f32),
        [pltpu.SemaphoreType.DMA] * NBUF,
        [pltpu.SemaphoreType.DMA] * NBUF,
        pltpu.VMEM_SHARED((N, HF), _f32),
    ],
    compiler_params=_sc_params,
)


BN = 1000
GRID = N // BN


def _embed_body(x_ref, w_ref, b_ref, o_ref):
    o_ref[...] = jnp.dot(x_ref[...], w_ref[...],
                         preferred_element_type=_f32) + b_ref[...]


def _embed(x, W, b):
    return pl.pallas_call(
        _embed_body,
        grid=(GRID,),
        in_specs=[
            pl.BlockSpec((BN, F), lambda i: (i, 0)),
            pl.BlockSpec((F, F), lambda i: (0, 0)),
            pl.BlockSpec((1, F), lambda i: (0, 0)),
        ],
        out_specs=pl.BlockSpec((BN, F), lambda i: (i, 0)),
        out_shape=jax.ShapeDtypeStruct((N, F), _f32),
    )(x, W, b.reshape(1, F))


def _split_store(hs_ref, v):
    hs_ref[0, :, :] = v[:, :HF]
    hs_ref[1, :, :] = v[:, HF:]


def _scale_body(h_ref, d_ref, hs_ref, io_ref, ii_ref):
    d = d_ref[...]
    do = d[0, :, 0:1] + d[2, :, 0:1]
    di = d[1, :, 0:1] + d[3, :, 0:1]
    io = jnp.where(do > 0, lax.rsqrt(do), 0.0)
    ii = jnp.where(di > 0, lax.rsqrt(di), 0.0)
    _split_store(hs_ref, h_ref[...] * io)
    io_ref[...] = io
    ii_ref[...] = ii


def _scale(h, deg):
    return pl.pallas_call(
        _scale_body,
        grid=(GRID,),
        in_specs=[
            pl.BlockSpec((BN, F), lambda i: (i, 0)),
            pl.BlockSpec((4, BN, 16), lambda i: (0, i, 0)),
        ],
        out_specs=[
            pl.BlockSpec((2, BN, HF), lambda i: (0, i, 0)),
            pl.BlockSpec((BN, 1), lambda i: (i, 0)),
            pl.BlockSpec((BN, 1), lambda i: (i, 0)),
        ],
        out_shape=[
            jax.ShapeDtypeStruct((2, N, HF), _f32),
            jax.ShapeDtypeStruct((N, 1), _f32),
            jax.ShapeDtypeStruct((N, 1), _f32),
        ],
    )(h, deg)


def _layer_body(p_ref, ii_ref, sc_ref, w_ref, b_ref, o_ref):
    p = p_ref[...]
    m = jnp.concatenate([p[0], p[1]], axis=1) * ii_ref[...]
    h = jnp.maximum(jnp.dot(m, w_ref[...], preferred_element_type=_f32)
                    + b_ref[...], 0.0)
    _split_store(o_ref, h * sc_ref[...])


def _layer(p, ii, scale, W, b):
    return pl.pallas_call(
        _layer_body,
        grid=(GRID,),
        in_specs=[
            pl.BlockSpec((2, BN, HF), lambda i: (0, i, 0)),
            pl.BlockSpec((BN, 1), lambda i: (i, 0)),
            pl.BlockSpec((BN, 1), lambda i: (i, 0)),
            pl.BlockSpec((F, F), lambda i: (0, 0)),
            pl.BlockSpec((1, F), lambda i: (0, 0)),
        ],
        out_specs=pl.BlockSpec((2, BN, HF), lambda i: (0, i, 0)),
        out_shape=jax.ShapeDtypeStruct((2, N, HF), _f32),
    )(p, ii, scale, W, b.reshape(1, F))


def _head_body(h_ref, w1_ref, b1_ref, w2_ref, b2_ref, w3_ref, b3_ref,
               o_ref, acc_ref):
    i = pl.program_id(0)
    hb = h_ref[...]
    s = jnp.concatenate(
        [jnp.sum(hb[0], axis=0, keepdims=True),
         jnp.sum(hb[1], axis=0, keepdims=True)], axis=1)

    @pl.when(i == 0)
    def _():
        acc_ref[...] = s

    @pl.when(i > 0)
    def _():
        acc_ref[...] = acc_ref[...] + s

    @pl.when(i == GRID - 1)
    def _():
        g = acc_ref[...] * (1.0 / N)
        o = jnp.maximum(jnp.dot(g, w1_ref[...],
                                preferred_element_type=_f32) + b1_ref[...], 0.0)
        o = jnp.maximum(jnp.dot(o, w2_ref[...],
                                preferred_element_type=_f32) + b2_ref[...], 0.0)
        o_ref[...] = jnp.dot(o, w3_ref[...],
                             preferred_element_type=_f32) + b3_ref[...]


def _head(h, Wo1, bo1, Wo2, bo2, Wo3, bo3):
    H2, H4 = F // 2, F // 4
    return pl.pallas_call(
        _head_body,
        grid=(GRID,),
        in_specs=[
            pl.BlockSpec((2, BN, HF), lambda i: (0, i, 0)),
            pl.BlockSpec((F, H2), lambda i: (0, 0)),
            pl.BlockSpec((1, H2), lambda i: (0, 0)),
            pl.BlockSpec((H2, H4), lambda i: (0, 0)),
            pl.BlockSpec((1, H4), lambda i: (0, 0)),
            pl.BlockSpec((H4, 1), lambda i: (0, 0)),
            pl.BlockSpec((1, 1), lambda i: (0, 0)),
        ],
        out_specs=pl.BlockSpec((1, 1), lambda i: (0, 0)),
        out_shape=jax.ShapeDtypeStruct((1, 1), _f32),
        scratch_shapes=[pltpu.VMEM((1, F), _f32)],
    )(h, Wo1, bo1.reshape(1, H2), Wo2, bo2.reshape(1, H4),
      Wo3, bo3.reshape(1, 1))


def kernel(x, edge_index, W_atom, b_atom, W1, b1, W2, b2, W3, b3,
           Wo1, bo1, Wo2, bo2, Wo3, bo3):
    ei = edge_index.astype(jnp.int32)
    e3d = ei.reshape(2, NW, K, C)
    e3a = ei.reshape(2, NS, K2, CA)
    ones16 = jnp.ones((C, 16), _f32)
    z16 = jnp.zeros((N, 16), _f32)
    zrow = jnp.zeros((N, HF), _f32)

    deg = _deg_call(e3d, ones16, z16).reshape(4, N, 16)
    h = _embed(x, W_atom, b_atom)
    hs, io, ii = _scale(h, deg)

    ones_n = jnp.ones((N, 1), _f32)
    for W, b, sc in ((W1, b1, io), (W2, b2, io), (W3, b3, ones_n)):
        p = _agg_call(hs, e3a, zrow)
        hs = _layer(p, ii, sc, W, b)

    out = _head(hs, Wo1, bo1, Wo2, bo2, Wo3, bo3)
    return out.reshape(1)


# speedup vs baseline: 10.6778x; 1.1541x over previous
import functools

import jax
import jax.numpy as jnp
from jax import lax
from jax.experimental import pallas as pl
from jax.experimental.pallas import tpu as pltpu
from jax.experimental.pallas import tpu_sc as plsc

N = 10000
E = 320000
F = 128

NC = 2
NS = 16
NW = NC * NS
C = 100
K = E // (NW * C)
RPT = N // NS
ZR = 125

_f32 = jnp.float32
_mesh = plsc.VectorSubcoreMesh(core_axis_name="c", subcore_axis_name="s")
_sc_params = pltpu.CompilerParams(use_tc_tiling_on_sc=False)


def _wid(c, s):
    return c * NS + s


def _deg_body(edge_hbm, ones_hbm, z_hbm, out_hbm,
              src_v, dst_v, ones_v, degs_sh, degd_sh):
    c = lax.axis_index("c")
    s = lax.axis_index("s")
    w = _wid(c, s)

    pltpu.sync_copy(edge_hbm.at[0, w], src_v)
    pltpu.sync_copy(edge_hbm.at[1, w], dst_v)
    pltpu.sync_copy(ones_hbm, ones_v)

    @pl.when(s == 0)
    def _():
        pltpu.sync_copy(z_hbm, degs_sh)
        pltpu.sync_copy(z_hbm, degd_sh)

    plsc.subcore_barrier()

    def body(j, carry):
        pltpu.sync_copy(ones_v, degs_sh.at[src_v.at[j]], add=True)
        pltpu.sync_copy(ones_v, degd_sh.at[dst_v.at[j]], add=True)
        return carry

    lax.fori_loop(0, K, body, 0)
    plsc.subcore_barrier()

    @pl.when(s == 0)
    def _():
        pltpu.sync_copy(degs_sh, out_hbm.at[c, 0])
        pltpu.sync_copy(degd_sh, out_hbm.at[c, 1])


_deg_call = pl.kernel(
    _deg_body,
    out_type=jax.ShapeDtypeStruct((2, 2, N, 16), _f32),
    mesh=_mesh,
    scratch_types=[
        pltpu.VMEM((K, C), jnp.int32),
        pltpu.VMEM((K, C), jnp.int32),
        pltpu.VMEM((C, 16), _f32),
        pltpu.VMEM_SHARED((N, 16), _f32),
        pltpu.VMEM_SHARED((N, 16), _f32),
    ],
    compiler_params=_sc_params,
)


HF = F // 2
CA = 125
K2 = E // (NS * CA)
NBUF = 5
HALF = NBUF // 2


def _agg_body(hs_hbm, edge_hbm, z_hbm, out_hbm,
              src_v, dst_v, rows, gsem, ssem, acc_sh):
    c = lax.axis_index("c")
    s = lax.axis_index("s")
    hsc = hs_hbm.at[c]

    pltpu.sync_copy(edge_hbm.at[0, s], src_v)
    pltpu.sync_copy(edge_hbm.at[1, s], dst_v)

    @pl.when(s == 0)
    def _():
        pltpu.sync_copy(z_hbm, acc_sh)

    for b in range(NBUF):
        pltpu.async_copy(hsc.at[src_v.at[b]], rows.at[b], gsem[b])
    plsc.subcore_barrier()

    def chunk(j, b):
        pltpu.make_async_copy(hsc.at[src_v.at[j]], rows.at[b], gsem[b]).wait()
        pltpu.async_copy(rows.at[b], acc_sh.at[dst_v.at[j]], ssem[b],
                         add=True)
        bq = (b - HALF) % NBUF

        @pl.when(jnp.logical_and(j >= HALF, j - HALF + NBUF < K2))
        def _():
            pltpu.make_async_copy(rows.at[bq], acc_sh.at[dst_v.at[0]],
                                  ssem[bq]).wait()
            pltpu.async_copy(hsc.at[src_v.at[j - HALF + NBUF]], rows.at[bq],
                             gsem[bq])

    def body(t, carry):
        j = t * NBUF
        for b in range(NBUF):
            chunk(j + b, b)
        return carry

    lax.fori_loop(0, K2 // NBUF, body, 0)
    for b in range(NBUF):
        pltpu.make_async_copy(rows.at[b], acc_sh.at[dst_v.at[0]],
                              ssem[b]).wait()
    plsc.subcore_barrier()

    @pl.when(s == 0)
    def _():
        pltpu.sync_copy(acc_sh, out_hbm.at[c])


_agg_call = pl.kernel(
    _agg_body,
    out_type=jax.ShapeDtypeStruct((2, N, HF), _f32),
    mesh=_mesh,
    scratch_types=[
        pltpu.VMEM((K2, CA), jnp.int32),
        pltpu.VMEM((K2, CA), jnp.int32),
        pltpu.VMEM((NBUF, CA, HF), _f32),
        [pltpu.SemaphoreType.DMA] * NBUF,
        [pltpu.SemaphoreType.DMA] * NBUF,
        pltpu.VMEM_SHARED((N, HF), _f32),
    ],
    compiler_params=_sc_params,
)


BN = 1000
GRID = N // BN


def _split_store(hs_ref, v):
    hs_ref[0, :, :] = v[:, :HF]
    hs_ref[1, :, :] = v[:, HF:]


def _embed_body(x_ref, d_ref, w_ref, b_ref, hs_ref, io_ref, ii_ref):
    d = d_ref[...]
    do = d[0, :, 0:1] + d[2, :, 0:1]
    di = d[1, :, 0:1] + d[3, :, 0:1]
    io = jnp.where(do > 0, lax.rsqrt(do), 0.0)
    ii = jnp.where(di > 0, lax.rsqrt(di), 0.0)
    h = jnp.dot(x_ref[...], w_ref[...], preferred_element_type=_f32) \
        + b_ref[...]
    _split_store(hs_ref, h * io)
    io_ref[...] = io
    ii_ref[...] = ii


def _embed_scale(x, deg, W, b):
    return pl.pallas_call(
        _embed_body,
        grid=(GRID,),
        in_specs=[
            pl.BlockSpec((BN, F), lambda i: (i, 0)),
            pl.BlockSpec((4, BN, 16), lambda i: (0, i, 0)),
            pl.BlockSpec((F, F), lambda i: (0, 0)),
            pl.BlockSpec((1, F), lambda i: (0, 0)),
        ],
        out_specs=[
            pl.BlockSpec((2, BN, HF), lambda i: (0, i, 0)),
            pl.BlockSpec((BN, 1), lambda i: (i, 0)),
            pl.BlockSpec((BN, 1), lambda i: (i, 0)),
        ],
        out_shape=[
            jax.ShapeDtypeStruct((2, N, HF), _f32),
            jax.ShapeDtypeStruct((N, 1), _f32),
            jax.ShapeDtypeStruct((N, 1), _f32),
        ],
    )(x, deg, W, b.reshape(1, F))


def _layer_body(p_ref, ii_ref, sc_ref, w_ref, b_ref, o_ref):
    p = p_ref[...]
    m = jnp.concatenate([p[0], p[1]], axis=1) * ii_ref[...]
    h = jnp.maximum(jnp.dot(m, w_ref[...], preferred_element_type=_f32)
                    + b_ref[...], 0.0)
    _split_store(o_ref, h * sc_ref[...])


def _layer(p, ii, scale, W, b):
    return pl.pallas_call(
        _layer_body,
        grid=(GRID,),
        in_specs=[
            pl.BlockSpec((2, BN, HF), lambda i: (0, i, 0)),
            pl.BlockSpec((BN, 1), lambda i: (i, 0)),
            pl.BlockSpec((BN, 1), lambda i: (i, 0)),
            pl.BlockSpec((F, F), lambda i: (0, 0)),
            pl.BlockSpec((1, F), lambda i: (0, 0)),
        ],
        out_specs=pl.BlockSpec((2, BN, HF), lambda i: (0, i, 0)),
        out_shape=jax.ShapeDtypeStruct((2, N, HF), _f32),
    )(p, ii, scale, W, b.reshape(1, F))


def _last_body(p_ref, ii_ref, w_ref, b_ref, w1_ref, b1_ref, w2_ref, b2_ref,
               w3_ref, b3_ref, o_ref, acc_ref):
    i = pl.program_id(0)
    p = p_ref[...]
    m = jnp.concatenate([p[0], p[1]], axis=1) * ii_ref[...]
    h = jnp.maximum(jnp.dot(m, w_ref[...], preferred_element_type=_f32)
                    + b_ref[...], 0.0)
    s = jnp.sum(h, axis=0, keepdims=True)

    @pl.when(i == 0)
    def _():
        acc_ref[...] = s

    @pl.when(i > 0)
    def _():
        acc_ref[...] = acc_ref[...] + s

    @pl.when(i == GRID - 1)
    def _():
        g = acc_ref[...] * (1.0 / N)
        o = jnp.maximum(jnp.dot(g, w1_ref[...],
                                preferred_element_type=_f32) + b1_ref[...], 0.0)
        o = jnp.maximum(jnp.dot(o, w2_ref[...],
                                preferred_element_type=_f32) + b2_ref[...], 0.0)
        o_ref[...] = jnp.dot(o, w3_ref[...],
                             preferred_element_type=_f32) + b3_ref[...]


def _last_layer_head(p, ii, W, b, Wo1, bo1, Wo2, bo2, Wo3, bo3):
    H2, H4 = F // 2, F // 4
    return pl.pallas_call(
        _last_body,
        grid=(GRID,),
        in_specs=[
            pl.BlockSpec((2, BN, HF), lambda i: (0, i, 0)),
            pl.BlockSpec((BN, 1), lambda i: (i, 0)),
            pl.BlockSpec((F, F), lambda i: (0, 0)),
            pl.BlockSpec((1, F), lambda i: (0, 0)),
            pl.BlockSpec((F, H2), lambda i: (0, 0)),
            pl.BlockSpec((1, H2), lambda i: (0, 0)),
            pl.BlockSpec((H2, H4), lambda i: (0, 0)),
            pl.BlockSpec((1, H4), lambda i: (0, 0)),
            pl.BlockSpec((H4, 1), lambda i: (0, 0)),
            pl.BlockSpec((1, 1), lambda i: (0, 0)),
        ],
        out_specs=pl.BlockSpec((1, 1), lambda i: (0, 0)),
        out_shape=jax.ShapeDtypeStruct((1, 1), _f32),
        scratch_shapes=[pltpu.VMEM((1, F), _f32)],
    )(p, ii, W, b.reshape(1, F), Wo1, bo1.reshape(1, H2),
      Wo2, bo2.reshape(1, H4), Wo3, bo3.reshape(1, 1))


def kernel(x, edge_index, W_atom, b_atom, W1, b1, W2, b2, W3, b3,
           Wo1, bo1, Wo2, bo2, Wo3, bo3):
    ei = edge_index.astype(jnp.int32)
    e3d = ei.reshape(2, NW, K, C)
    e3a = ei.reshape(2, NS, K2, CA)
    ones16 = jnp.ones((C, 16), _f32)
    z16 = jnp.zeros((N, 16), _f32)
    zrow = jnp.zeros((N, HF), _f32)

    deg = _deg_call(e3d, ones16, z16).reshape(4, N, 16)
    hs, io, ii = _embed_scale(x, deg, W_atom, b_atom)

    for W, b in ((W1, b1), (W2, b2)):
        p = _agg_call(hs, e3a, zrow)
        hs = _layer(p, ii, io, W, b)

    p = _agg_call(hs, e3a, zrow)
    out = _last_layer_head(p, ii, W3, b3, Wo1, bo1, Wo2, bo2, Wo3, bo3)
    return out.reshape(1)


# speedup vs baseline: 13.4974x; 1.2641x over previous
import functools

import jax
import jax.numpy as jnp
from jax import lax
from jax.experimental import pallas as pl
from jax.experimental.pallas import tpu as pltpu
from jax.experimental.pallas import tpu_sc as plsc

N = 10000
E = 320000
F = 128

NC = 2
NS = 16
NW = NC * NS
C = 100
K = E // (NW * C)
RPT = N // NS
ZR = 125

_f32 = jnp.float32
_bf16 = jnp.bfloat16
_mesh = plsc.VectorSubcoreMesh(core_axis_name="c", subcore_axis_name="s")
_sc_params = pltpu.CompilerParams(use_tc_tiling_on_sc=False)


def _wid(c, s):
    return c * NS + s


def _deg_body(edge_hbm, ones_hbm, z_hbm, out_hbm,
              src_v, dst_v, ones_v, degs_sh, degd_sh, sem):
    c = lax.axis_index("c")
    s = lax.axis_index("s")
    w = _wid(c, s)
    rz = N // NS

    pltpu.sync_copy(edge_hbm.at[0, w], src_v)
    pltpu.sync_copy(edge_hbm.at[1, w], dst_v)
    pltpu.sync_copy(ones_hbm, ones_v)
    pltpu.sync_copy(z_hbm, degs_sh.at[pl.ds(s * rz, rz)])
    pltpu.sync_copy(z_hbm, degd_sh.at[pl.ds(s * rz, rz)])
    plsc.subcore_barrier()

    def body(j, carry):
        pltpu.async_copy(ones_v, degs_sh.at[src_v.at[j]], sem, add=True)
        pltpu.async_copy(ones_v, degd_sh.at[dst_v.at[j]], sem, add=True)
        return carry

    lax.fori_loop(0, K, body, 0)

    def drain(j, carry):
        pltpu.make_async_copy(ones_v, degs_sh.at[src_v.at[0]], sem).wait()
        pltpu.make_async_copy(ones_v, degd_sh.at[dst_v.at[0]], sem).wait()
        return carry

    lax.fori_loop(0, K, drain, 0)
    plsc.subcore_barrier()

    pltpu.sync_copy(degs_sh.at[pl.ds(s * rz, rz)],
                    out_hbm.at[c, 0, pl.ds(s * rz, rz)])
    pltpu.sync_copy(degd_sh.at[pl.ds(s * rz, rz)],
                    out_hbm.at[c, 1, pl.ds(s * rz, rz)])


_deg_call = pl.kernel(
    _deg_body,
    out_type=jax.ShapeDtypeStruct((2, 2, N, 16), _f32),
    mesh=_mesh,
    scratch_types=[
        pltpu.VMEM((K, C), jnp.int32),
        pltpu.VMEM((K, C), jnp.int32),
        pltpu.VMEM((C, 16), _f32),
        pltpu.VMEM_SHARED((N, 16), _f32),
        pltpu.VMEM_SHARED((N, 16), _f32),
        pltpu.SemaphoreType.DMA,
    ],
    compiler_params=_sc_params,
)


HF = F // 2
CA = 125
K2 = E // (NS * CA)
NBUF = 5
HALF = NBUF // 2


def _agg_body(hs_hbm, edge_hbm, z_hbm, out_hbm,
              src_v, dst_v, rows, gsem, ssem, acc_sh):
    c = lax.axis_index("c")
    s = lax.axis_index("s")
    hsc = hs_hbm.at[c]

    rz = N // NS
    pltpu.sync_copy(edge_hbm.at[0, s], src_v)
    pltpu.sync_copy(edge_hbm.at[1, s], dst_v)
    pltpu.sync_copy(z_hbm, acc_sh.at[pl.ds(s * rz, rz)])

    for b in range(NBUF):
        pltpu.async_copy(hsc.at[src_v.at[b]], rows.at[b], gsem[b])
    plsc.subcore_barrier()

    def chunk(j, b):
        pltpu.make_async_copy(hsc.at[src_v.at[j]], rows.at[b], gsem[b]).wait()
        pltpu.async_copy(rows.at[b], acc_sh.at[dst_v.at[j]], ssem[b],
                         add=True)
        bq = (b - HALF) % NBUF

        @pl.when(jnp.logical_and(j >= HALF, j - HALF + NBUF < K2))
        def _():
            pltpu.make_async_copy(rows.at[bq], acc_sh.at[dst_v.at[0]],
                                  ssem[bq]).wait()
            pltpu.async_copy(hsc.at[src_v.at[j - HALF + NBUF]], rows.at[bq],
                             gsem[bq])

    def body(t, carry):
        j = t * NBUF
        for b in range(NBUF):
            chunk(j + b, b)
        return carry

    lax.fori_loop(0, K2 // NBUF, body, 0)
    for b in range(NBUF):
        pltpu.make_async_copy(rows.at[b], acc_sh.at[dst_v.at[0]],
                              ssem[b]).wait()
    plsc.subcore_barrier()
    pltpu.sync_copy(acc_sh.at[pl.ds(s * rz, rz)],
                    out_hbm.at[c, pl.ds(s * rz, rz)])


_agg_call = pl.kernel(
    _agg_body,
    out_type=jax.ShapeDtypeStruct((2, N, HF), _bf16),
    mesh=_mesh,
    scratch_types=[
        pltpu.VMEM((K2, CA), jnp.int32),
        pltpu.VMEM((K2, CA), jnp.int32),
        pltpu.VMEM((NBUF, CA, HF), _bf16),
        [pltpu.SemaphoreType.DMA] * NBUF,
        [pltpu.SemaphoreType.DMA] * NBUF,
        pltpu.VMEM_SHARED((N, HF), _bf16),
    ],
    compiler_params=_sc_params,
)


BN = 1000
GRID = N // BN


def _split_store(hs_ref, v):
    v = v.astype(_bf16)
    hs_ref[0, :, :] = v[:, :HF]
    hs_ref[1, :, :] = v[:, HF:]


def _embed_body(x_ref, d_ref, w_ref, b_ref, hs_ref, io_ref, ii_ref):
    d = d_ref[...]
    do = d[0, :, 0:1] + d[2, :, 0:1]
    di = d[1, :, 0:1] + d[3, :, 0:1]
    io = jnp.where(do > 0, lax.rsqrt(do), 0.0)
    ii = jnp.where(di > 0, lax.rsqrt(di), 0.0)
    h = jnp.dot(x_ref[...], w_ref[...], preferred_element_type=_f32) \
        + b_ref[...]
    _split_store(hs_ref, h * io)
    io_ref[...] = io
    ii_ref[...] = ii


def _embed_scale(x, deg, W, b):
    return pl.pallas_call(
        _embed_body,
        grid=(GRID,),
        in_specs=[
            pl.BlockSpec((BN, F), lambda i: (i, 0)),
            pl.BlockSpec((4, BN, 16), lambda i: (0, i, 0)),
            pl.BlockSpec((F, F), lambda i: (0, 0)),
            pl.BlockSpec((1, F), lambda i: (0, 0)),
        ],
        out_specs=[
            pl.BlockSpec((2, BN, HF), lambda i: (0, i, 0)),
            pl.BlockSpec((BN, 1), lambda i: (i, 0)),
            pl.BlockSpec((BN, 1), lambda i: (i, 0)),
        ],
        out_shape=[
            jax.ShapeDtypeStruct((2, N, HF), _bf16),
            jax.ShapeDtypeStruct((N, 1), _f32),
            jax.ShapeDtypeStruct((N, 1), _f32),
        ],
    )(x, deg, W, b.reshape(1, F))


def _layer_body(p_ref, ii_ref, sc_ref, w_ref, b_ref, o_ref):
    p = p_ref[...].astype(_f32)
    m = jnp.concatenate([p[0], p[1]], axis=1) * ii_ref[...]
    h = jnp.maximum(jnp.dot(m, w_ref[...], preferred_element_type=_f32)
                    + b_ref[...], 0.0)
    _split_store(o_ref, h * sc_ref[...])


def _layer(p, ii, scale, W, b):
    return pl.pallas_call(
        _layer_body,
        grid=(GRID,),
        in_specs=[
            pl.BlockSpec((2, BN, HF), lambda i: (0, i, 0)),
            pl.BlockSpec((BN, 1), lambda i: (i, 0)),
            pl.BlockSpec((BN, 1), lambda i: (i, 0)),
            pl.BlockSpec((F, F), lambda i: (0, 0)),
            pl.BlockSpec((1, F), lambda i: (0, 0)),
        ],
        out_specs=pl.BlockSpec((2, BN, HF), lambda i: (0, i, 0)),
        out_shape=jax.ShapeDtypeStruct((2, N, HF), _bf16),
    )(p, ii, scale, W, b.reshape(1, F))


def _last_body(p_ref, ii_ref, w_ref, b_ref, w1_ref, b1_ref, w2_ref, b2_ref,
               w3_ref, b3_ref, o_ref, acc_ref):
    i = pl.program_id(0)
    p = p_ref[...].astype(_f32)
    m = jnp.concatenate([p[0], p[1]], axis=1) * ii_ref[...]
    h = jnp.maximum(jnp.dot(m, w_ref[...], preferred_element_type=_f32)
                    + b_ref[...], 0.0)
    s = jnp.sum(h, axis=0, keepdims=True)

    @pl.when(i == 0)
    def _():
        acc_ref[...] = s

    @pl.when(i > 0)
    def _():
        acc_ref[...] = acc_ref[...] + s

    @pl.when(i == GRID - 1)
    def _():
        g = acc_ref[...] * (1.0 / N)
        o = jnp.maximum(jnp.dot(g, w1_ref[...],
                                preferred_element_type=_f32) + b1_ref[...], 0.0)
        o = jnp.maximum(jnp.dot(o, w2_ref[...],
                                preferred_element_type=_f32) + b2_ref[...], 0.0)
        o_ref[...] = jnp.dot(o, w3_ref[...],
                             preferred_element_type=_f32) + b3_ref[...]


def _last_layer_head(p, ii, W, b, Wo1, bo1, Wo2, bo2, Wo3, bo3):
    H2, H4 = F // 2, F // 4
    return pl.pallas_call(
        _last_body,
        grid=(GRID,),
        in_specs=[
            pl.BlockSpec((2, BN, HF), lambda i: (0, i, 0)),
            pl.BlockSpec((BN, 1), lambda i: (i, 0)),
            pl.BlockSpec((F, F), lambda i: (0, 0)),
            pl.BlockSpec((1, F), lambda i: (0, 0)),
            pl.BlockSpec((F, H2), lambda i: (0, 0)),
            pl.BlockSpec((1, H2), lambda i: (0, 0)),
            pl.BlockSpec((H2, H4), lambda i: (0, 0)),
            pl.BlockSpec((1, H4), lambda i: (0, 0)),
            pl.BlockSpec((H4, 1), lambda i: (0, 0)),
            pl.BlockSpec((1, 1), lambda i: (0, 0)),
        ],
        out_specs=pl.BlockSpec((1, 1), lambda i: (0, 0)),
        out_shape=jax.ShapeDtypeStruct((1, 1), _f32),
        scratch_shapes=[pltpu.VMEM((1, F), _f32)],
    )(p, ii, W, b.reshape(1, F), Wo1, bo1.reshape(1, H2),
      Wo2, bo2.reshape(1, H4), Wo3, bo3.reshape(1, 1))


def kernel(x, edge_index, W_atom, b_atom, W1, b1, W2, b2, W3, b3,
           Wo1, bo1, Wo2, bo2, Wo3, bo3):
    ei = edge_index.astype(jnp.int32)
    e3d = ei.reshape(2, NW, K, C)
    e3a = ei.reshape(2, NS, K2, CA)
    ones16 = jnp.ones((C, 16), _f32)
    z16 = jnp.zeros((N // NS, 16), _f32)
    zrow = jnp.zeros((N // NS, HF), _bf16)

    deg = _deg_call(e3d, ones16, z16).reshape(4, N, 16)
    hs, io, ii = _embed_scale(x, deg, W_atom, b_atom)

    for W, b in ((W1, b1), (W2, b2)):
        p = _agg_call(hs, e3a, zrow)
        hs = _layer(p, ii, io, W, b)

    p = _agg_call(hs, e3a, zrow)
    out = _last_layer_head(p, ii, W3, b3, Wo1, bo1, Wo2, bo2, Wo3, bo3)
    return out.reshape(1)
